# baseline (speedup 1.0000x reference)
import jax
import jax.numpy as jnp
from jax import lax
from jax.experimental import pallas as pl
from jax.experimental.pallas import tpu as pltpu
from jax.experimental.pallas import tpu_sc as plsc

VOCAB = 100000
EMBED_DIM = 128
BATCH = 4096
HIST = 200

NC, NS = 2, 16
NW = NC * NS
TOTAL = BATCH * HIST
ROWS_PER_W = TOTAL // NW
CHUNK = 128
GATHERS = CHUNK // 128
STEPS = ROWS_PER_W // CHUNK
NBUF = 4


def _embed_kernel(idx_hbm, table_hbm, out_hbm, *refs):
    wid = lax.axis_index("s") * NC + lax.axis_index("c")
    def row0(c):
        return (c * NW + wid) * CHUNK
    idx_bufs = refs[0:NBUF]
    rows_bufs = refs[NBUF:2 * NBUF]
    gsems = refs[2 * NBUF:3 * NBUF]
    osems = refs[3 * NBUF:4 * NBUF]
    isems = refs[4 * NBUF:5 * NBUF]

    def fire_gathers(c, b):
        for j in range(GATHERS):
            pltpu.async_copy(
                table_hbm.at[idx_bufs[b].at[pl.ds(j * 128, 128)]],
                rows_bufs[b].at[pl.ds(j * 128, 128)],
                gsems[b],
            )

    def fire_idx(c, b):
        pltpu.async_copy(
            idx_hbm.at[pl.ds(row0(c), CHUNK)], idx_bufs[b], isems[b]
        )

    def drain_gathers(b):
        for j in range(GATHERS):
            pltpu.make_async_copy(
                table_hbm.at[idx_bufs[b].at[pl.ds(j * 128, 128)]],
                rows_bufs[b].at[pl.ds(j * 128, 128)],
                gsems[b],
            ).wait()

    def drain_idx(b):
        pltpu.make_async_copy(
            idx_hbm.at[pl.ds(0, CHUNK)], idx_bufs[b], isems[b]
        ).wait()

    def drain_out(b):
        pltpu.make_async_copy(
            rows_bufs[b], out_hbm.at[pl.ds(0, CHUNK)], osems[b]
        ).wait()

    for b in range(NBUF):
        pltpu.sync_copy(idx_hbm.at[pl.ds(row0(b), CHUNK)], idx_bufs[b])
        fire_gathers(b, b)

    def superstep(s, carry):
        c0 = s * NBUF
        for b in range(NBUF):
            drain_gathers(b)
            pltpu.async_copy(
                rows_bufs[b],
                out_hbm.at[pl.ds(row0(c0 + b), CHUNK)],
                osems[b],
            )
            fire_idx(c0 + b + NBUF, b)
        for b in range(NBUF):
            drain_idx(b)
            drain_out(b)
            fire_gathers(c0 + b + NBUF, b)
        return carry

    lax.fori_loop(0, STEPS // NBUF - 1, superstep, 0)

    for b in range(NBUF):
        drain_gathers(b)
        pltpu.async_copy(
            rows_bufs[b],
            out_hbm.at[pl.ds(row0(STEPS - NBUF + b), CHUNK)],
            osems[b],
        )
    for b in range(NBUF):
        drain_out(b)


@jax.jit
def kernel(sentence_seqs, word_embeds):
    idx = sentence_seqs.reshape(TOTAL).astype(jnp.int32)
    mesh = plsc.VectorSubcoreMesh(core_axis_name="c", subcore_axis_name="s")
    out = pl.kernel(
        _embed_kernel,
        out_type=jax.ShapeDtypeStruct((TOTAL, EMBED_DIM), jnp.float32),
        mesh=mesh,
        scratch_types=(
            [pltpu.VMEM((CHUNK,), jnp.int32)] * NBUF
            + [pltpu.VMEM((CHUNK, EMBED_DIM), jnp.float32)] * NBUF
            + [pltpu.SemaphoreType.DMA] * (3 * NBUF)
        ),
    )(idx, word_embeds)
    return out.reshape(BATCH, HIST, EMBED_DIM)


# speedup vs baseline: 1.0026x; 1.0026x over previous
import jax
import jax.numpy as jnp
from jax import lax
from jax.experimental import pallas as pl
from jax.experimental.pallas import tpu as pltpu
from jax.experimental.pallas import tpu_sc as plsc

VOCAB = 100000
EMBED_DIM = 128
BATCH = 4096
HIST = 200

NC, NS = 2, 16
NW = NC * NS
TOTAL = BATCH * HIST
ROWS_PER_W = TOTAL // NW
CHUNK = 128
GATHERS = CHUNK // 128
STEPS = ROWS_PER_W // CHUNK
NBUF = 5


def _embed_kernel(idx_hbm, table_hbm, out_hbm, *refs):
    wid = lax.axis_index("s") * NC + lax.axis_index("c")
    def row0(c):
        return (c * NW + wid) * CHUNK
    idx_bufs = refs[0:NBUF]
    rows_bufs = refs[NBUF:2 * NBUF]
    gsems = refs[2 * NBUF:3 * NBUF]
    osems = refs[3 * NBUF:4 * NBUF]
    isems = refs[4 * NBUF:5 * NBUF]

    def fire_gathers(c, b):
        for j in range(GATHERS):
            pltpu.async_copy(
                table_hbm.at[idx_bufs[b].at[pl.ds(j * 128, 128)]],
                rows_bufs[b].at[pl.ds(j * 128, 128)],
                gsems[b],
            )

    def fire_idx(c, b):
        pltpu.async_copy(
            idx_hbm.at[pl.ds(row0(c), CHUNK)], idx_bufs[b], isems[b]
        )

    def drain_gathers(b):
        for j in range(GATHERS):
            pltpu.make_async_copy(
                table_hbm.at[idx_bufs[b].at[pl.ds(j * 128, 128)]],
                rows_bufs[b].at[pl.ds(j * 128, 128)],
                gsems[b],
            ).wait()

    def drain_idx(b):
        pltpu.make_async_copy(
            idx_hbm.at[pl.ds(0, CHUNK)], idx_bufs[b], isems[b]
        ).wait()

    def drain_out(b):
        pltpu.make_async_copy(
            rows_bufs[b], out_hbm.at[pl.ds(0, CHUNK)], osems[b]
        ).wait()

    for b in range(NBUF):
        pltpu.sync_copy(idx_hbm.at[pl.ds(row0(b), CHUNK)], idx_bufs[b])
        fire_gathers(b, b)

    def superstep(s, carry):
        c0 = s * NBUF
        for b in range(NBUF):
            drain_gathers(b)
            pltpu.async_copy(
                rows_bufs[b],
                out_hbm.at[pl.ds(row0(c0 + b), CHUNK)],
                osems[b],
            )
            fire_idx(c0 + b + NBUF, b)
        for b in range(NBUF):
            drain_idx(b)
            drain_out(b)
            fire_gathers(c0 + b + NBUF, b)
        return carry

    lax.fori_loop(0, STEPS // NBUF - 1, superstep, 0)

    for b in range(NBUF):
        drain_gathers(b)
        pltpu.async_copy(
            rows_bufs[b],
            out_hbm.at[pl.ds(row0(STEPS - NBUF + b), CHUNK)],
            osems[b],
        )
    for b in range(NBUF):
        drain_out(b)


@jax.jit
def kernel(sentence_seqs, word_embeds):
    idx = sentence_seqs.reshape(TOTAL).astype(jnp.int32)
    mesh = plsc.VectorSubcoreMesh(core_axis_name="c", subcore_axis_name="s")
    out = pl.kernel(
        _embed_kernel,
        out_type=jax.ShapeDtypeStruct((TOTAL, EMBED_DIM), jnp.float32),
        mesh=mesh,
        scratch_types=(
            [pltpu.VMEM((CHUNK,), jnp.int32)] * NBUF
            + [pltpu.VMEM((CHUNK, EMBED_DIM), jnp.float32)] * NBUF
            + [pltpu.SemaphoreType.DMA] * (3 * NBUF)
        ),
    )(idx, word_embeds)
    return out.reshape(BATCH, HIST, EMBED_DIM)


# speedup vs baseline: 1.0054x; 1.0028x over previous
import jax
import jax.numpy as jnp
from jax import lax
from jax.experimental import pallas as pl
from jax.experimental.pallas import tpu as pltpu
from jax.experimental.pallas import tpu_sc as plsc

VOCAB = 100000
EMBED_DIM = 128
BATCH = 4096
HIST = 200

NC, NS = 2, 16
NW = NC * NS
TOTAL = BATCH * HIST
ROWS_PER_W = TOTAL // NW
CHUNK = 64
GATHERS = max(1, CHUNK // 128)
STEPS = ROWS_PER_W // CHUNK
NBUF = 8


def _embed_kernel(idx_hbm, table_hbm, out_hbm, *refs):
    wid = lax.axis_index("s") * NC + lax.axis_index("c")
    def row0(c):
        return (c * NW + wid) * CHUNK
    idx_bufs = refs[0:NBUF]
    rows_bufs = refs[NBUF:2 * NBUF]
    gsems = refs[2 * NBUF:3 * NBUF]
    osems = refs[3 * NBUF:4 * NBUF]
    isems = refs[4 * NBUF:5 * NBUF]

    def fire_gathers(c, b):
        for j in range(GATHERS):
            pltpu.async_copy(
                table_hbm.at[idx_bufs[b].at[pl.ds(j * CHUNK, CHUNK)]],
                rows_bufs[b].at[pl.ds(j * CHUNK, CHUNK)],
                gsems[b],
            )

    def fire_idx(c, b):
        pltpu.async_copy(
            idx_hbm.at[pl.ds(row0(c), CHUNK)], idx_bufs[b], isems[b]
        )

    def drain_gathers(b):
        for j in range(GATHERS):
            pltpu.make_async_copy(
                table_hbm.at[idx_bufs[b].at[pl.ds(j * CHUNK, CHUNK)]],
                rows_bufs[b].at[pl.ds(j * CHUNK, CHUNK)],
                gsems[b],
            ).wait()

    def drain_idx(b):
        pltpu.make_async_copy(
            idx_hbm.at[pl.ds(0, CHUNK)], idx_bufs[b], isems[b]
        ).wait()

    def drain_out(b):
        pltpu.make_async_copy(
            rows_bufs[b], out_hbm.at[pl.ds(0, CHUNK)], osems[b]
        ).wait()

    for b in range(NBUF):
        pltpu.sync_copy(idx_hbm.at[pl.ds(row0(b), CHUNK)], idx_bufs[b])
        fire_gathers(b, b)

    def superstep(s, carry):
        c0 = s * NBUF
        for b in range(NBUF):
            drain_gathers(b)
            pltpu.async_copy(
                rows_bufs[b],
                out_hbm.at[pl.ds(row0(c0 + b), CHUNK)],
                osems[b],
            )
            fire_idx(c0 + b + NBUF, b)
        for b in range(NBUF):
            drain_idx(b)
            drain_out(b)
            fire_gathers(c0 + b + NBUF, b)
        return carry

    lax.fori_loop(0, STEPS // NBUF - 1, superstep, 0)

    for b in range(NBUF):
        drain_gathers(b)
        pltpu.async_copy(
            rows_bufs[b],
            out_hbm.at[pl.ds(row0(STEPS - NBUF + b), CHUNK)],
            osems[b],
        )
    for b in range(NBUF):
        drain_out(b)


@jax.jit
def kernel(sentence_seqs, word_embeds):
    idx = sentence_seqs.reshape(TOTAL).astype(jnp.int32)
    mesh = plsc.VectorSubcoreMesh(core_axis_name="c", subcore_axis_name="s")
    out = pl.kernel(
        _embed_kernel,
        out_type=jax.ShapeDtypeStruct((TOTAL, EMBED_DIM), jnp.float32),
        mesh=mesh,
        scratch_types=(
            [pltpu.VMEM((CHUNK,), jnp.int32)] * NBUF
            + [pltpu.VMEM((CHUNK, EMBED_DIM), jnp.float32)] * NBUF
            + [pltpu.SemaphoreType.DMA] * (3 * NBUF)
        ),
    )(idx, word_embeds)
    return out.reshape(BATCH, HIST, EMBED_DIM)
